# baseline (speedup 1.0000x reference)
import functools

import jax
import jax.numpy as jnp
from jax import lax
from jax.experimental import pallas as pl
from jax.experimental.pallas import tpu as pltpu
from jax.experimental.pallas import tpu_sc as plsc

_INFO = plsc.get_sparse_core_info()
_NC = _INFO.num_cores
_NS = _INFO.num_subcores
_NW = _NC * _NS

_ROWS = 8192
_D = 1024
_ROWS_W = _ROWS // _NW
_C = 16
_NCH = _ROWS_W // _C
_VPR = _D // 16


def _make_sc_call():
    mesh = plsc.VectorSubcoreMesh(core_axis_name="c", subcore_axis_name="s")

    @functools.partial(
        pl.kernel,
        out_type=jax.ShapeDtypeStruct((4, _ROWS // 4, _D), jnp.float32),
        mesh=mesh,
        scratch_types=[
            pltpu.VMEM((_ROWS_W,), jnp.int32),
            [pltpu.VMEM((_C, _D), jnp.float32) for _ in range(2)],
            [pltpu.VMEM((_C, _D), jnp.float32) for _ in range(2)],
            [pltpu.SemaphoreType.DMA for _ in range(2)],
            [pltpu.SemaphoreType.DMA for _ in range(2)],
            [pltpu.SemaphoreType.DMA for _ in range(2)],
        ],
    )
    def sc_call(x_hbm, idx_hbm, pe_hbm, out_hbm, idx_v, buf_x, buf_pe,
                sem_x, sem_pe, sem_out):
        wid = lax.axis_index("s") * _NC + lax.axis_index("c")
        nwb = _NW // 4
        b = wid // nwb
        base = (wid % nwb) * _ROWS_W
        idx_cp = pltpu.async_copy(
            idx_hbm.at[b, pl.ds(base, _ROWS_W)], idx_v, sem_pe[1])

        def start_loads(j, p):
            row0 = base + j * _C
            pltpu.async_copy(x_hbm.at[b, pl.ds(row0, _C)], buf_x[p], sem_x[p])
            pltpu.async_copy(
                pe_hbm.at[idx_v.at[pl.ds(j * _C, _C)]], buf_pe[p], sem_pe[p])

        def wait_loads(j, p):
            row0 = base + j * _C
            pltpu.make_async_copy(
                x_hbm.at[b, pl.ds(row0, _C)], buf_x[p], sem_x[p]).wait()
            pltpu.make_async_copy(
                pe_hbm.at[idx_v.at[pl.ds(j * _C, _C)]], buf_pe[p],
                sem_pe[p]).wait()


        def step(j, p):
            wait_loads(j, p)

            @plsc.parallel_loop(0, _C, 1, unroll=2)
            def add_row(r):
                for v in range(_VPR):
                    sl = (r, pl.ds(v * 16, 16))
                    plsc.addupdate(buf_x[p].at[sl], buf_pe[p][sl])

            pltpu.async_copy(
                buf_x[p], out_hbm.at[b, pl.ds(base + j * _C, _C)], sem_out[p])

        def wait_out(j, p):
            pltpu.make_async_copy(
                buf_x[p], out_hbm.at[b, pl.ds(base + j * _C, _C)],
                sem_out[p]).wait()

        pltpu.async_copy(x_hbm.at[b, pl.ds(base, _C)], buf_x[0], sem_x[0])
        idx_cp.wait()
        pltpu.async_copy(pe_hbm.at[idx_v.at[pl.ds(0, _C)]], buf_pe[0],
                         sem_pe[0])

        def body(j2, carry):
            j = j2 * 2
            @pl.when(j2 > 0)
            def _():
                wait_out(j - 1, 1)
            start_loads(j + 1, 1)
            step(j, 0)
            wait_out(j, 0)

            @pl.when(j2 < _NCH // 2 - 1)
            def _():
                start_loads(j + 2, 0)
            step(j + 1, 1)
            return carry

        lax.fori_loop(0, _NCH // 2, body, 0)
        wait_out(_NCH - 1, 1)

    return sc_call


_SC_CALL = _make_sc_call()


def kernel(x, pids, pe):
    return _SC_CALL(x, pids.astype(jnp.int32), pe)


# speedup vs baseline: 1.1747x; 1.1747x over previous
import functools

import jax
import jax.numpy as jnp
from jax import lax
from jax.experimental import pallas as pl
from jax.experimental.pallas import tpu as pltpu
from jax.experimental.pallas import tpu_sc as plsc

_INFO = plsc.get_sparse_core_info()
_NC = _INFO.num_cores
_NS = _INFO.num_subcores
_NW = _NC * _NS

_ROWS = 8192
_D = 1024
_ROWS_W = _ROWS // _NW
_C = 16
_NCH = _ROWS_W // _C
_VPR = _D // 16


def _make_sc_call():
    mesh = plsc.VectorSubcoreMesh(core_axis_name="c", subcore_axis_name="s")

    @functools.partial(
        pl.kernel,
        out_type=jax.ShapeDtypeStruct((4, _ROWS // 4, _D), jnp.float32),
        mesh=mesh,
        scratch_types=[
            pltpu.VMEM((_ROWS_W,), jnp.int32),
            [pltpu.VMEM((_C, _D), jnp.float32) for _ in range(2)],
            [pltpu.VMEM((_C, _D), jnp.float32) for _ in range(2)],
            [pltpu.SemaphoreType.DMA for _ in range(2)],
            [pltpu.SemaphoreType.DMA for _ in range(2)],
            [pltpu.SemaphoreType.DMA for _ in range(2)],
        ],
    )
    def sc_call(x_hbm, idx_hbm, pe_hbm, out_hbm, idx_v, buf_x, buf_pe,
                sem_x, sem_pe, sem_out):
        wid = lax.axis_index("s") * _NC + lax.axis_index("c")
        nwb = _NW // 4
        b = wid // nwb
        base = (wid % nwb) * _ROWS_W
        idx_cp = pltpu.async_copy(
            idx_hbm.at[b, pl.ds(base, _ROWS_W)], idx_v, sem_pe[1])

        def start_loads(j, p):
            row0 = base + j * _C
            pltpu.async_copy(x_hbm.at[b, pl.ds(row0, _C)], buf_x[p], sem_x[p])
            pltpu.async_copy(
                pe_hbm.at[idx_v.at[pl.ds(j * _C, _C)]], buf_pe[p], sem_pe[p])

        def wait_loads(j, p):
            row0 = base + j * _C
            pltpu.make_async_copy(
                x_hbm.at[b, pl.ds(row0, _C)], buf_x[p], sem_x[p]).wait()
            pltpu.make_async_copy(
                pe_hbm.at[idx_v.at[pl.ds(j * _C, _C)]], buf_pe[p],
                sem_pe[p]).wait()


        def step(j, p):
            wait_loads(j, p)

            @plsc.parallel_loop(0, _C, 1)
            def add_row(r):
                for v in range(_VPR):
                    sl = (r, pl.ds(v * 16, 16))
                    buf_x[p][sl] = buf_x[p][sl] + buf_pe[p][sl]

            pltpu.async_copy(
                buf_x[p], out_hbm.at[b, pl.ds(base + j * _C, _C)], sem_out[p])

        def wait_out(j, p):
            pltpu.make_async_copy(
                buf_x[p], out_hbm.at[b, pl.ds(base + j * _C, _C)],
                sem_out[p]).wait()

        pltpu.async_copy(x_hbm.at[b, pl.ds(base, _C)], buf_x[0], sem_x[0])
        idx_cp.wait()
        pltpu.async_copy(pe_hbm.at[idx_v.at[pl.ds(0, _C)]], buf_pe[0],
                         sem_pe[0])

        def body(j2, carry):
            j = j2 * 2
            @pl.when(j2 > 0)
            def _():
                wait_out(j - 1, 1)
            start_loads(j + 1, 1)
            step(j, 0)
            wait_out(j, 0)

            @pl.when(j2 < _NCH // 2 - 1)
            def _():
                start_loads(j + 2, 0)
            step(j + 1, 1)
            return carry

        lax.fori_loop(0, _NCH // 2, body, 0)
        wait_out(_NCH - 1, 1)

    return sc_call


_SC_CALL = _make_sc_call()


def kernel(x, pids, pe):
    return _SC_CALL(x, pids.astype(jnp.int32), pe)
